# baseline (speedup 1.0000x reference)
import jax
import jax.numpy as jnp
from jax import lax
from jax.experimental import pallas as pl
from jax.experimental.pallas import tpu as pltpu
from jax.experimental.pallas import tpu_sc as plsc

NC = 2
NS = 16
NW = NC * NS

GBUF = 4
OBUF = 2
LANES = 16
BB = 128


def _make_sc_kernel(B, S, D, scale):
    nvec = D // LANES
    drows = D // 8

    mesh = plsc.VectorSubcoreMesh(core_axis_name="c", subcore_axis_name="s")

    def body(x_hbm, table_hbm, pe_hbm, out_hbm, idx_v, gbuf, obuf, pe_v,
             gsem, ssem):
        ci = lax.axis_index("c")
        si = lax.axis_index("s")
        w = si * NC + ci

        pltpu.sync_copy(x_hbm.at[w], idx_v)
        pltpu.sync_copy(pe_hbm.at[pl.ds(0, S)], pe_v)

        def start_gather(s, g):
            pltpu.async_copy(table_hbm.at[idx_v.at[s]], gbuf.at[g], gsem.at[g])

        def wait_gather(g):
            pltpu.make_async_copy(
                table_hbm.at[pl.ds(0, BB)], gbuf.at[g], gsem.at[g]
            ).wait()

        def start_write(s, o):
            pltpu.async_copy(
                obuf.at[pl.ds(o * 8, 8), :, pl.ds(0, BB)],
                out_hbm.at[s, :, w],
                ssem.at[o],
            )

        def wait_write(o):
            pltpu.make_async_copy(
                obuf.at[pl.ds(o * 8, 8), :, pl.ds(0, BB)],
                out_hbm.at[0, :, 0],
                ssem.at[o],
            ).wait()

        start_gather(0, 0)
        start_gather(1, 1)

        def s_body(s, g, o):
            @pl.when(s >= 2)
            def _():
                wait_write(o)

            @pl.when(s + 2 < S)
            def _():
                start_gather(s + 2, (g + 2) % GBUF)

            wait_gather(g)

            pvecs = [pe_v[s, pl.ds(j * LANES, LANES)] for j in range(nvec)]
            lanes = jnp.arange(LANES, dtype=jnp.int32)
            didx = [lanes + j * LANES for j in range(nvec)]
            rt_i = [d // 8 for d in didx]
            d8_i = [d % 8 for d in didx]
            zero = lanes * 0

            @plsc.parallel_loop(0, BB, step=1, unroll=4)
            def _(r):
                rv = zero + r
                for j in range(nvec):
                    v = gbuf[g, r, pl.ds(j * LANES, LANES)]
                    val = v + pvecs[j]
                    plsc.store_scatter(obuf, [rt_i[j] + o * 8, d8_i[j], rv], val)

            start_write(s, o)

        def outer(t, carry):
            s0 = t * GBUF
            for k in range(GBUF):
                s_body(s0 + k, k, k % OBUF)
            return carry

        lax.fori_loop(0, S // GBUF, outer, 0)
        wait_write(0)
        wait_write(1)

    return pl.kernel(
        body,
        out_type=jax.ShapeDtypeStruct((S, drows, NW, 8, BB), jnp.float32),
        mesh=mesh,
        scratch_types=[
            pltpu.VMEM((S, BB), jnp.int32),
            pltpu.VMEM((GBUF, BB, BB), jnp.float32),
            pltpu.VMEM((OBUF * 8, 8, BB + 1), jnp.float32),
            pltpu.VMEM((S, D), jnp.float32),
            pltpu.SemaphoreType.DMA((GBUF,)),
            pltpu.SemaphoreType.DMA((OBUF,)),
        ],
        compiler_params=pltpu.CompilerParams(
            use_tc_tiling_on_sc=False, needs_layout_passes=False
        ),
    )


def kernel(x, table, pos_encoding):
    B, S = x.shape
    V, D = table.shape
    scale = float(D) ** 0.5
    assert B == NW * BB
    assert D % LANES == 0 and S % GBUF == 0 and S % OBUF == 0

    xr = x.astype(jnp.int32).reshape(NW, BB, S).transpose((0, 2, 1))
    table_p = jnp.pad(table * scale, ((0, 0), (0, 128 - D)))
    out5 = _make_sc_kernel(B, S, D, scale)(xr, table_p, pos_encoding)
    return out5.transpose((2, 4, 0, 1, 3)).reshape(B, S, D)


# speedup vs baseline: 1.4105x; 1.4105x over previous
import jax
import jax.numpy as jnp
from jax import lax
from jax.experimental import pallas as pl
from jax.experimental.pallas import tpu as pltpu
from jax.experimental.pallas import tpu_sc as plsc

NC = 2
NS = 16
NW = NC * NS

GBUF = 4
OBUF = 2
LANES = 16
BB = 128


def _make_sc_kernel(B, S, D, scale):
    nvec = D // LANES
    drows = D // 8

    mesh = plsc.VectorSubcoreMesh(core_axis_name="c", subcore_axis_name="s")

    def body(x_hbm, table_hbm, pe_hbm, out_hbm, idx_v, gbuf, obuf, pe_v,
             gsem, ssem):
        ci = lax.axis_index("c")
        si = lax.axis_index("s")
        w = si * NC + ci

        pltpu.sync_copy(x_hbm.at[w], idx_v)
        pltpu.sync_copy(pe_hbm.at[pl.ds(0, S)], pe_v)

        def start_gather(s, g):
            pltpu.async_copy(table_hbm.at[idx_v.at[s]], gbuf.at[g], gsem.at[g])

        def wait_gather(g):
            pltpu.make_async_copy(
                table_hbm.at[pl.ds(0, BB)], gbuf.at[g], gsem.at[g]
            ).wait()

        def start_write(s, o):
            pltpu.async_copy(
                obuf.at[pl.ds(o * 8, 8), :, pl.ds(0, BB)],
                out_hbm.at[s, :, w],
                ssem.at[o],
            )

        def wait_write(o):
            pltpu.make_async_copy(
                obuf.at[pl.ds(o * 8, 8), :, pl.ds(0, BB)],
                out_hbm.at[0, :, 0],
                ssem.at[o],
            ).wait()

        start_gather(0, 0)
        start_gather(1, 1)

        def s_body(s, g, o):
            @pl.when(s >= 2)
            def _():
                wait_write(o)

            @pl.when(s + 2 < S)
            def _():
                start_gather(s + 2, (g + 2) % GBUF)

            wait_gather(g)

            pvecs = [pe_v[s, pl.ds(j * LANES, LANES)] for j in range(nvec)]
            lanes = jnp.arange(LANES, dtype=jnp.int32)
            didx = [lanes + j * LANES for j in range(nvec)]
            rt_i = [d // 8 for d in didx]
            d8_i = [d % 8 for d in didx]
            zero = lanes * 0

            @plsc.parallel_loop(0, BB, step=1, unroll=4)
            def _(r):
                rv = zero + r
                for j in range(nvec):
                    v = gbuf[g, r, pl.ds(j * LANES, LANES)]
                    val = v * scale + pvecs[j]
                    plsc.store_scatter(obuf, [rt_i[j] + o * 8, d8_i[j], rv], val)

            start_write(s, o)

        def outer(t, carry):
            s0 = t * GBUF
            for k in range(GBUF):
                s_body(s0 + k, k, k % OBUF)
            return carry

        lax.fori_loop(0, S // GBUF, outer, 0)
        wait_write(0)
        wait_write(1)

    return pl.kernel(
        body,
        out_type=jax.ShapeDtypeStruct((S, drows, NW, 8, BB), jnp.float32),
        mesh=mesh,
        scratch_types=[
            pltpu.VMEM((S, BB), jnp.int32),
            pltpu.VMEM((GBUF, BB, BB), jnp.float32),
            pltpu.VMEM((OBUF * 8, 8, BB + 1), jnp.float32),
            pltpu.VMEM((S, D), jnp.float32),
            pltpu.SemaphoreType.DMA((GBUF,)),
            pltpu.SemaphoreType.DMA((OBUF,)),
        ],
        compiler_params=pltpu.CompilerParams(
            use_tc_tiling_on_sc=False, needs_layout_passes=False
        ),
    )


def kernel(x, table, pos_encoding):
    B, S = x.shape
    V, D = table.shape
    scale = float(D) ** 0.5
    assert B == NW * BB
    assert D % LANES == 0 and S % GBUF == 0 and S % OBUF == 0

    xr = x.astype(jnp.int32).reshape(NW, BB, S).transpose((0, 2, 1))
    table_p = jnp.pad(table, ((0, 0), (0, 128 - D)))
    out5 = _make_sc_kernel(B, S, D, scale)(xr, table_p, pos_encoding)
    return out5.transpose((2, 4, 0, 1, 3)).reshape(B, S, D)


# speedup vs baseline: 1.4147x; 1.0030x over previous
import jax
import jax.numpy as jnp
from jax import lax
from jax.experimental import pallas as pl
from jax.experimental.pallas import tpu as pltpu
from jax.experimental.pallas import tpu_sc as plsc

NC = 2
NS = 16
NW = NC * NS

GBUF = 4
OBUF = 2
LANES = 16
BB = 128


def _make_sc_kernel(B, S, D, scale):
    nvec = D // LANES
    drows = D // 8

    mesh = plsc.VectorSubcoreMesh(core_axis_name="c", subcore_axis_name="s")

    def body(x_hbm, table_hbm, pe_hbm, out_hbm, idx_v, gbuf, obuf, pe_v,
             gsem, ssem):
        ci = lax.axis_index("c")
        si = lax.axis_index("s")
        w = si * NC + ci

        pltpu.sync_copy(x_hbm.at[w], idx_v)
        pltpu.sync_copy(pe_hbm.at[pl.ds(0, S)], pe_v)

        def start_gather(s, g):
            pltpu.async_copy(table_hbm.at[idx_v.at[s]], gbuf.at[g], gsem.at[g])

        def wait_gather(g):
            pltpu.make_async_copy(
                table_hbm.at[pl.ds(0, BB)], gbuf.at[g], gsem.at[g]
            ).wait()

        def start_write(s, o):
            pltpu.async_copy(
                obuf.at[pl.ds(o * 8, 8), :, pl.ds(0, BB)],
                out_hbm.at[s, :, w],
                ssem.at[o],
            )

        def wait_write(o):
            pltpu.make_async_copy(
                obuf.at[pl.ds(o * 8, 8), :, pl.ds(0, BB)],
                out_hbm.at[0, :, 0],
                ssem.at[o],
            ).wait()

        start_gather(0, 0)
        start_gather(1, 1)
        start_gather(2, 2)

        def s_body(s, g, o):
            @pl.when(s >= 2)
            def _():
                wait_write(o)

            @pl.when(s + 3 < S)
            def _():
                start_gather(s + 3, (g + 3) % GBUF)

            wait_gather(g)

            pvecs = [pe_v[s, pl.ds(j * LANES, LANES)] for j in range(nvec)]
            lanes = jnp.arange(LANES, dtype=jnp.int32)
            didx = [lanes + j * LANES for j in range(nvec)]
            rt_i = [d // 8 for d in didx]
            d8_i = [d % 8 for d in didx]
            zero = lanes * 0

            @plsc.parallel_loop(0, BB, step=1, unroll=4)
            def _(r):
                rv = zero + r
                for j in range(nvec):
                    v = gbuf[g, r, pl.ds(j * LANES, LANES)]
                    val = v * scale + pvecs[j]
                    plsc.store_scatter(obuf, [rt_i[j] + o * 8, d8_i[j], rv], val)

            start_write(s, o)

        def outer(t, carry):
            s0 = t * GBUF
            for k in range(GBUF):
                s_body(s0 + k, k, k % OBUF)
            return carry

        lax.fori_loop(0, S // GBUF, outer, 0)
        wait_write(0)
        wait_write(1)

    return pl.kernel(
        body,
        out_type=jax.ShapeDtypeStruct((S, drows, NW, 8, BB), jnp.float32),
        mesh=mesh,
        scratch_types=[
            pltpu.VMEM((S, BB), jnp.int32),
            pltpu.VMEM((GBUF, BB, BB), jnp.float32),
            pltpu.VMEM((OBUF * 8, 8, BB + 1), jnp.float32),
            pltpu.VMEM((S, D), jnp.float32),
            pltpu.SemaphoreType.DMA((GBUF,)),
            pltpu.SemaphoreType.DMA((OBUF,)),
        ],
        compiler_params=pltpu.CompilerParams(
            use_tc_tiling_on_sc=False, needs_layout_passes=False
        ),
    )


def kernel(x, table, pos_encoding):
    B, S = x.shape
    V, D = table.shape
    scale = float(D) ** 0.5
    assert B == NW * BB
    assert D % LANES == 0 and S % GBUF == 0 and S % OBUF == 0

    xr = x.astype(jnp.int32).reshape(NW, BB, S).transpose((0, 2, 1))
    table_p = jnp.pad(table, ((0, 0), (0, 128 - D)))
    out5 = _make_sc_kernel(B, S, D, scale)(xr, table_p, pos_encoding)
    return out5.transpose((2, 4, 0, 1, 3)).reshape(B, S, D)
